# Initial kernel scaffold; baseline (speedup 1.0000x reference)
import jax
import jax.numpy as jnp
from jax.experimental import pallas as pl


def kernel(logits, labels):
    raise NotImplementedError("write your pallas kernel here")


# speedup vs baseline: 1.1175x; 1.1175x over previous
import functools

import jax
import jax.numpy as jnp
from jax.experimental import pallas as pl

S = 64.0
MEAN = 0.35
SIGMA = 0.0125

ROWS = 1024
ROW_BLOCK = 256
COL_BLOCK = 6400


def _body(elastic_ref, labels_ref, logits_ref, out_ref):
    j = pl.program_id(1)
    col0 = j * COL_BLOCK
    cols = col0 + jax.lax.broadcasted_iota(jnp.int32, (ROW_BLOCK, COL_BLOCK), 1)
    lab = labels_ref[:, :]
    esc = elastic_ref[:, :]
    hit = cols == lab
    out_ref[:, :] = logits_ref[:, :] * S - jnp.where(hit, esc, 0.0)


def kernel(logits, labels):
    n_rows, n_cols = logits.shape
    ekey = jax.random.key(42)
    elastic_s = (MEAN + SIGMA * jax.random.normal(ekey, (n_rows,), dtype=jnp.float32)) * S
    elastic_s = elastic_s.reshape(n_rows, 1)
    labels2 = labels.reshape(n_rows, 1)

    grid = (n_rows // ROW_BLOCK, pl.cdiv(n_cols, COL_BLOCK))
    return pl.pallas_call(
        _body,
        grid=grid,
        in_specs=[
            pl.BlockSpec((ROW_BLOCK, 1), lambda i, j: (i, 0)),
            pl.BlockSpec((ROW_BLOCK, 1), lambda i, j: (i, 0)),
            pl.BlockSpec((ROW_BLOCK, COL_BLOCK), lambda i, j: (i, j)),
        ],
        out_specs=pl.BlockSpec((ROW_BLOCK, COL_BLOCK), lambda i, j: (i, j)),
        out_shape=jax.ShapeDtypeStruct((n_rows, n_cols), jnp.float32),
    )(elastic_s, labels2, logits)


# speedup vs baseline: 1.1195x; 1.0017x over previous
import functools

import jax
import jax.numpy as jnp
from jax.experimental import pallas as pl

S = 64.0
MEAN = 0.35
SIGMA = 0.0125

ROWS = 1024
ROW_BLOCK = 128
COL_BLOCK = 12800


def _body(elastic_ref, labels_ref, logits_ref, out_ref):
    j = pl.program_id(1)
    col0 = j * COL_BLOCK
    cols = col0 + jax.lax.broadcasted_iota(jnp.int32, (ROW_BLOCK, COL_BLOCK), 1)
    lab = labels_ref[:, :]
    esc = elastic_ref[:, :]
    hit = cols == lab
    out_ref[:, :] = logits_ref[:, :] * S - jnp.where(hit, esc, 0.0)


def kernel(logits, labels):
    n_rows, n_cols = logits.shape
    ekey = jax.random.key(42)
    elastic_s = (MEAN + SIGMA * jax.random.normal(ekey, (n_rows,), dtype=jnp.float32)) * S
    elastic_s = elastic_s.reshape(n_rows, 1)
    labels2 = labels.reshape(n_rows, 1)

    grid = (n_rows // ROW_BLOCK, pl.cdiv(n_cols, COL_BLOCK))
    return pl.pallas_call(
        _body,
        grid=grid,
        in_specs=[
            pl.BlockSpec((ROW_BLOCK, 1), lambda i, j: (i, 0)),
            pl.BlockSpec((ROW_BLOCK, 1), lambda i, j: (i, 0)),
            pl.BlockSpec((ROW_BLOCK, COL_BLOCK), lambda i, j: (i, j)),
        ],
        out_specs=pl.BlockSpec((ROW_BLOCK, COL_BLOCK), lambda i, j: (i, j)),
        out_shape=jax.ShapeDtypeStruct((n_rows, n_cols), jnp.float32),
    )(elastic_s, labels2, logits)
